# baseline (speedup 1.0000x reference)
import functools
import math

import jax
import jax.numpy as jnp
from jax import lax
from jax.experimental import pallas as pl
from jax.experimental.pallas import tpu as pltpu
from jax.experimental.pallas import tpu_sc as plsc

N = 512
H = 512
K = 16
ND = 3
G = 4
NEG = -1e30
BIG = 3.0e38

F32 = jnp.float32
I32 = jnp.int32

_NC = 2
_NS = 16
_NW = _NC * _NS
_RPW = (G * N) // _NW
_NCHUNK = N // 16


def _dot(a, b):
    return jax.lax.dot_general(a, b, (((1,), (0,)), ((), ())),
                               preferred_element_type=F32)


def _dot_t(a, b):
    return jax.lax.dot_general(a, b, (((1,), (1,)), ((), ())),
                               preferred_element_type=F32)


def _dot_bf(a, b):
    return jax.lax.dot_general(a.astype(jnp.bfloat16),
                               b.astype(jnp.bfloat16),
                               (((1,), (0,)), ((), ())),
                               preferred_element_type=F32)


def _layernorm(x, scale, bias, eps=1e-6):
    mu = jnp.mean(x, axis=-1, keepdims=True)
    var = jnp.mean(jnp.square(x - mu), axis=-1, keepdims=True)
    return (x - mu) * jax.lax.rsqrt(var + eps) * scale + bias


def _sc_knn_body(posT_hbm, out_hbm, pos_v, ibuf_v, dbuf_v, sbuf_v, sibuf_v):
    wid = lax.axis_index("s") * _NC + lax.axis_index("c")
    g = wid // (N // _RPW)
    nbase = (wid % (N // _RPW)) * _RPW

    pltpu.sync_copy(posT_hbm.at[g], pos_v)

    iota16 = lax.iota(I32, 16)
    _gdn = lax.GatherDimensionNumbers(offset_dims=(), collapsed_slice_dims=(0,),
                                      start_index_map=(0,))

    def _bcast_lane(vec, lane):
        return lax.gather(vec, lane[:, None], _gdn, (1,),
                          mode=lax.GatherScatterMode.PROMISE_IN_BOUNDS)

    def merge(a, b):
        ak, ai = a
        bk, bi = b
        rbk = lax.rev(bk, (0,))
        rbi = lax.rev(bi, (0,))
        take = ak <= rbk
        mk = jnp.where(take, ak, rbk)
        mi = jnp.where(take, ai, rbi)
        return lax.sort((mk, mi), num_keys=1)

    def one_row(n, dbuf_v, sbuf_v, sibuf_v):
        cb = (n // 16) * 16
        lane = jnp.full((16,), n - cb, I32)
        pxn = _bcast_lane(pos_v[pl.ds(cb, 16)], lane)
        pyn = _bcast_lane(pos_v[pl.ds(N + cb, 16)], lane)
        pzn = _bcast_lane(pos_v[pl.ds(2 * N + cb, 16)], lane)

        lmin = jnp.full((16,), BIG, F32)
        for c in range(_NCHUNK):
            dx = pos_v[pl.ds(c * 16, 16)] - pxn
            dy = pos_v[pl.ds(N + c * 16, 16)] - pyn
            dz = pos_v[pl.ds(2 * N + c * 16, 16)] - pzn
            d2 = dx * dx + dy * dy + dz * dz
            dbuf_v[pl.ds(c * 16, 16)] = d2
            lmin = jnp.minimum(lmin, d2)
        tv = jnp.full((16,), jnp.max(lmin), F32)

        def compact(c, cnt):
            d2 = dbuf_v[pl.ds(c * 16, 16)]
            m = d2 <= tv
            plsc.store_compressed(sbuf_v.at[pl.ds(cnt, 16)], d2, mask=m)
            plsc.store_compressed(sibuf_v.at[pl.ds(cnt, 16)],
                                  iota16 + c * 16, mask=m)
            return cnt + plsc.all_reduce_population_count(m)[0]
        cnt = 0
        for c in range(_NCHUNK):
            cnt = compact(c, cnt)

        best = lax.sort((sbuf_v[pl.ds(0, 16)], sibuf_v[pl.ds(0, 16)]),
                        num_keys=1)

        def mbody(c, carry):
            kk = sbuf_v[pl.ds(c * 16, 16)]
            ii = sibuf_v[pl.ds(c * 16, 16)]
            valid = (iota16 + c * 16) < cnt
            kk = jnp.where(valid, kk, BIG)
            return merge(carry, lax.sort((kk, ii), num_keys=1))

        nchunks = (cnt + 15) // 16
        best = lax.fori_loop(1, nchunks, mbody, best)
        return best[1]

    def row_body(r, _):
        n0 = nbase + r
        ibuf_v[pl.ds(r * 128, K)] = one_row(n0, dbuf_v, sbuf_v, sibuf_v)
        return 0

    lax.fori_loop(0, _RPW, row_body, 0)
    pltpu.sync_copy(ibuf_v, out_hbm.at[pl.ds(wid * (_RPW * 128), _RPW * 128)])


def _sc_knn(posT):
    mesh = plsc.VectorSubcoreMesh(core_axis_name="c", subcore_axis_name="s",
                                  num_cores=_NC, num_subcores=_NS)
    fn = pl.kernel(
        _sc_knn_body,
        out_type=jax.ShapeDtypeStruct((G * N * 128,), I32),
        mesh=mesh,
        compiler_params=pltpu.CompilerParams(needs_layout_passes=False),
        scratch_types=[
            pltpu.VMEM((ND * N,), F32),
            pltpu.VMEM((_RPW * 128,), I32),
            pltpu.VMEM((N,), F32),
            pltpu.VMEM((N + 32,), F32),
            pltpu.VMEM((N + 32,), I32),
        ],
    )
    return fn(posT)


def _pt_layer(x, pos, M, Wq, bq, Wk, bk, Wv, bv, Wpe, bpe, Wpd, bpd, wa, ba,
              Wo, bo, lns, lnb):
    q = _dot(x, Wq) + bq
    k = _dot(x, Wk) + bk
    v = _dot_bf(x, Wv) + bv
    pe = _dot(pos, Wpe) + bpe
    qq = q + pe
    qw = qq * wa
    kpe = k + pe
    u = _dot_t(qw, Wpd)
    c = (jnp.sum(qw * bpd, axis=1, keepdims=True) + ba
         - jnp.sum(u * pos, axis=1, keepdims=True))
    L = _dot_t(qw, kpe) + _dot_t(u, pos) + c
    Lm = jnp.where(M, L, NEG)
    rmax = jnp.max(Lm, axis=1, keepdims=True)
    e = jnp.where(M, jnp.exp(Lm - rmax), 0.0)
    A = e / jnp.sum(e, axis=1, keepdims=True)
    out = _dot_bf(A, v)
    y = jax.nn.relu(_dot_bf(out, Wo) + bo)
    x = x + y
    return _layernorm(x, lns, lnb)


def _main_kernel(pts_ref, idx_ref, *rest):
    W0, b0 = rest[0], rest[1]
    l0 = rest[2:18]
    l1 = rest[18:34]
    (eWk, ebk, eWq, ebq, eWv, ebv, eWo1, ebo1, eWo2, ebo2, elns, elnb
     ) = rest[34:46]
    out_ref, xp_ref = rest[46], rest[47]
    g = pl.program_id(0)

    @pl.when(g < G)
    def _cloud():
        pts = pts_ref[0]
        pos = pts[:, :ND]
        idx = idx_ref[0][:, :K]
        iota = jax.lax.broadcasted_iota(I32, (N, N), 1)
        M = jnp.zeros((N, N), jnp.bool_)
        for j in range(K):
            M = jnp.logical_or(M, iota == idx[:, j:j + 1])

        x = _dot(pts, W0[...]) + b0[...]
        x = _pt_layer(x, pos, M, *(w[...] for w in l0))
        x = _pt_layer(x, pos, M, *(w[...] for w in l1))
        xp_ref[pl.ds(g, 1), :] = jnp.max(x, axis=0, keepdims=True)

    @pl.when(g == G)
    def _enc():
        scale = 1.0 / math.sqrt(float(H))
        for b in range(G // 2):
            xb = xp_ref[2 * b:2 * b + 2, :]
            k = _dot(xb, eWk[...]) + ebk[...]
            q = _dot(xb, eWq[...]) + ebq[...]
            v = _dot_bf(xb, eWv[...]) + ebv[...]
            attn = _dot_t(q, k) * scale
            attn = attn - jnp.max(attn, axis=1, keepdims=True)
            e = jnp.exp(attn)
            attn = e / jnp.sum(e, axis=1, keepdims=True)
            out = _dot(attn, v)
            out = jax.nn.relu(_dot_bf(out, eWo1[...]) + ebo1[...])
            out = _dot_bf(out, eWo2[...]) + ebo2[...]
            xo = _layernorm(xb + out, elns[...], elnb[...])
            out_ref[b] = jnp.max(xo, axis=0)


def _row(a):
    return a.reshape(1, -1)


@jax.jit
def kernel(points, params):
    B, S, Np, C = points.shape
    pts = points.reshape(G, Np, C)
    posT = jnp.swapaxes(pts[..., :ND], 1, 2).reshape(G, ND * Np)

    nidx = _sc_knn(posT).reshape(G, Np, 128)

    p = params
    args = [p['W0'], _row(p['b0'])]
    for i in range(2):
        lp = p['layer%d' % i]
        args += [
            lp['Wq'], _row(lp['bq']),
            lp['Wk'], _row(lp['bk']),
            lp['Wv'].astype(jnp.bfloat16), _row(lp['bv']),
            lp['Wpe'], _row(lp['bpe']),
            lp['Wpd'], _row(lp['bpd']),
            lp['Wa'].reshape(1, H), lp['ba'].reshape(1, 1),
            lp['Wo'].astype(jnp.bfloat16), _row(lp['bo']),
            _row(lp['ln_scale']), _row(lp['ln_bias']),
        ]

    ep = p['enc']
    args += [ep['Wk'], _row(ep['bk']), ep['Wq'], _row(ep['bq']),
             ep['Wv'].astype(jnp.bfloat16), _row(ep['bv']),
             ep['Wo1'].astype(jnp.bfloat16), _row(ep['bo1']),
             ep['Wo2'].astype(jnp.bfloat16), _row(ep['bo2']),
             _row(ep['ln_scale']), _row(ep['ln_bias'])]

    rep = [pl.BlockSpec(a.shape, lambda g, nd=a.ndim: (0,) * nd) for a in args]
    clip = lambda g: (jnp.minimum(g, G - 1), 0, 0)
    out = pl.pallas_call(
        _main_kernel,
        grid=(G + 1,),
        in_specs=[
            pl.BlockSpec((1, Np, C), clip),
            pl.BlockSpec((1, Np, 128), clip),
        ] + rep,
        out_specs=pl.BlockSpec((B, H), lambda g: (0, 0)),
        out_shape=jax.ShapeDtypeStruct((B, H), F32),
        scratch_shapes=[pltpu.VMEM((G, H), F32)],
    )(pts, nidx, *args)
    return out


# speedup vs baseline: 1.1120x; 1.1120x over previous
import functools
import math

import jax
import jax.numpy as jnp
from jax import lax
from jax.experimental import pallas as pl
from jax.experimental.pallas import tpu as pltpu
from jax.experimental.pallas import tpu_sc as plsc

N = 512
H = 512
K = 16
ND = 3
G = 4
NEG = -1e30
BIG = 3.0e38

F32 = jnp.float32
I32 = jnp.int32

_NC = 2
_NS = 16
_NW = _NC * _NS
_RPW = (G * N) // _NW
_NCHUNK = N // 16


def _dot(a, b):
    return jax.lax.dot_general(a, b, (((1,), (0,)), ((), ())),
                               preferred_element_type=F32)


def _dot_t(a, b):
    return jax.lax.dot_general(a, b, (((1,), (1,)), ((), ())),
                               preferred_element_type=F32)


def _dot_bf(a, b):
    return jax.lax.dot_general(a.astype(jnp.bfloat16),
                               b.astype(jnp.bfloat16),
                               (((1,), (0,)), ((), ())),
                               preferred_element_type=F32)


def _layernorm(x, scale, bias, eps=1e-6):
    mu = jnp.mean(x, axis=-1, keepdims=True)
    var = jnp.mean(jnp.square(x - mu), axis=-1, keepdims=True)
    return (x - mu) * jax.lax.rsqrt(var + eps) * scale + bias


def _sc_knn_body(posT_hbm, out_hbm, pos_v, ibuf_v):
    wid = lax.axis_index("s") * _NC + lax.axis_index("c")
    g = wid // (N // _RPW)
    nbase = (wid % (N // _RPW)) * _RPW

    pltpu.sync_copy(posT_hbm.at[g], pos_v)

    iota16 = lax.iota(I32, 16)
    _gdn = lax.GatherDimensionNumbers(offset_dims=(), collapsed_slice_dims=(0,),
                                      start_index_map=(0,))

    def _bcast_lane(vec, lane):
        return lax.gather(vec, lane[:, None], _gdn, (1,),
                          mode=lax.GatherScatterMode.PROMISE_IN_BOUNDS)

    def one_row(n):
        cb = (n // 16) * 16
        lane = jnp.full((16,), n - cb, I32)
        pxn = _bcast_lane(pos_v[pl.ds(cb, 16)], lane)
        pyn = _bcast_lane(pos_v[pl.ds(N + cb, 16)], lane)
        pzn = _bcast_lane(pos_v[pl.ds(2 * N + cb, 16)], lane)

        def chunk_sorted(c):
            dx = pos_v[pl.ds(c * 16, 16)] - pxn
            dy = pos_v[pl.ds(N + c * 16, 16)] - pyn
            dz = pos_v[pl.ds(2 * N + c * 16, 16)] - pzn
            d2 = dx * dx + dy * dy + dz * dz
            return lax.sort((d2, iota16 + (c * 16)), num_keys=1)

        def merge(a, b):
            ak, ai = a
            bk, bi = b
            rbk = lax.rev(bk, (0,))
            rbi = lax.rev(bi, (0,))
            take = ak <= rbk
            mk = jnp.where(take, ak, rbk)
            mi = jnp.where(take, ai, rbi)
            return lax.sort((mk, mi), num_keys=1)

        def topk_range(c0, c1):
            if c1 - c0 == 1:
                return chunk_sorted(c0)
            mid = (c0 + c1) // 2
            return merge(topk_range(c0, mid), topk_range(mid, c1))

        _, bidx = topk_range(0, _NCHUNK)
        return bidx

    def row_body(r, _):
        ibuf_v[pl.ds(r * 128, K)] = one_row(nbase + r)
        return 0

    lax.fori_loop(0, _RPW, row_body, 0)
    pltpu.sync_copy(ibuf_v, out_hbm.at[pl.ds(wid * (_RPW * 128), _RPW * 128)])


def _sc_knn(posT):
    mesh = plsc.VectorSubcoreMesh(core_axis_name="c", subcore_axis_name="s",
                                  num_cores=_NC, num_subcores=_NS)
    fn = pl.kernel(
        _sc_knn_body,
        out_type=jax.ShapeDtypeStruct((G * N * 128,), I32),
        mesh=mesh,
        compiler_params=pltpu.CompilerParams(needs_layout_passes=False),
        scratch_types=[
            pltpu.VMEM((ND * N,), F32),
            pltpu.VMEM((_RPW * 128,), I32),
        ],
    )
    return fn(posT)


def _pt_layer(x, pos, M, Wq, bq, Wk, bk, Wv, bv, Wpe, bpe, Wpd, bpd, wa, ba,
              Wo, bo, lns, lnb):
    q = _dot(x, Wq) + bq
    k = _dot(x, Wk) + bk
    v = _dot_bf(x, Wv) + bv
    pe = _dot(pos, Wpe) + bpe
    qq = q + pe
    qw = qq * wa
    kpe = k + pe
    u = _dot_t(qw, Wpd)
    c = (jnp.sum(qw * bpd, axis=1, keepdims=True) + ba
         - jnp.sum(u * pos, axis=1, keepdims=True))
    L = _dot_t(qw, kpe) + _dot_t(u, pos) + c
    Lm = jnp.where(M, L, NEG)
    rmax = jnp.max(Lm, axis=1, keepdims=True)
    e = jnp.exp(Lm - rmax)
    A = e / jnp.sum(e, axis=1, keepdims=True)
    out = _dot_bf(A, v)
    y = jax.nn.relu(_dot_bf(out, Wo) + bo)
    x = x + y
    return _layernorm(x, lns, lnb)


def _main_kernel(pts_ref, idx_ref, *rest):
    W0, b0 = rest[0], rest[1]
    l0 = rest[2:18]
    l1 = rest[18:34]
    (eWk, ebk, eWq, ebq, eWv, ebv, eWo1, ebo1, eWo2, ebo2, elns, elnb
     ) = rest[34:46]
    out_ref, xp_ref = rest[46], rest[47]
    g = pl.program_id(0)

    @pl.when(g < G)
    def _cloud():
        pts = pts_ref[0]
        pos = pts[:, :ND]
        idx = idx_ref[0][:, :K].astype(jnp.int16)
        iota = jax.lax.broadcasted_iota(jnp.int16, (N, N), 1)
        M = jnp.zeros((N, N), jnp.bool_)
        for j in range(K):
            M = jnp.logical_or(M, iota == idx[:, j:j + 1])

        x = _dot(pts, W0[...]) + b0[...]
        x = _pt_layer(x, pos, M, *(w[...] for w in l0))
        x = _pt_layer(x, pos, M, *(w[...] for w in l1))
        xp_ref[pl.ds(g, 1), :] = jnp.max(x, axis=0, keepdims=True)

    @pl.when(g == G)
    def _enc():
        scale = 1.0 / math.sqrt(float(H))
        for b in range(G // 2):
            xb = xp_ref[2 * b:2 * b + 2, :]
            k = _dot(xb, eWk[...]) + ebk[...]
            q = _dot(xb, eWq[...]) + ebq[...]
            v = _dot_bf(xb, eWv[...]) + ebv[...]
            attn = _dot_t(q, k) * scale
            attn = attn - jnp.max(attn, axis=1, keepdims=True)
            e = jnp.exp(attn)
            attn = e / jnp.sum(e, axis=1, keepdims=True)
            out = _dot(attn, v)
            out = jax.nn.relu(_dot_bf(out, eWo1[...]) + ebo1[...])
            out = _dot_bf(out, eWo2[...]) + ebo2[...]
            xo = _layernorm(xb + out, elns[...], elnb[...])
            out_ref[b] = jnp.max(xo, axis=0)


def _row(a):
    return a.reshape(1, -1)


@jax.jit
def kernel(points, params):
    B, S, Np, C = points.shape
    pts = points.reshape(G, Np, C)
    posT = jnp.swapaxes(pts[..., :ND], 1, 2).reshape(G, ND * Np)

    nidx = _sc_knn(posT).reshape(G, Np, 128)

    p = params
    args = [p['W0'], _row(p['b0'])]
    for i in range(2):
        lp = p['layer%d' % i]
        args += [
            lp['Wq'], _row(lp['bq']),
            lp['Wk'], _row(lp['bk']),
            lp['Wv'].astype(jnp.bfloat16), _row(lp['bv']),
            lp['Wpe'], _row(lp['bpe']),
            lp['Wpd'], _row(lp['bpd']),
            lp['Wa'].reshape(1, H), lp['ba'].reshape(1, 1),
            lp['Wo'].astype(jnp.bfloat16), _row(lp['bo']),
            _row(lp['ln_scale']), _row(lp['ln_bias']),
        ]

    ep = p['enc']
    args += [ep['Wk'], _row(ep['bk']), ep['Wq'], _row(ep['bq']),
             ep['Wv'].astype(jnp.bfloat16), _row(ep['bv']),
             ep['Wo1'].astype(jnp.bfloat16), _row(ep['bo1']),
             ep['Wo2'].astype(jnp.bfloat16), _row(ep['bo2']),
             _row(ep['ln_scale']), _row(ep['ln_bias'])]

    rep = [pl.BlockSpec(a.shape, lambda g, nd=a.ndim: (0,) * nd) for a in args]
    clip = lambda g: (jnp.minimum(g, G - 1), 0, 0)
    out = pl.pallas_call(
        _main_kernel,
        grid=(G + 1,),
        in_specs=[
            pl.BlockSpec((1, Np, C), clip),
            pl.BlockSpec((1, Np, 128), clip),
        ] + rep,
        out_specs=pl.BlockSpec((B, H), lambda g: (0, 0)),
        out_shape=jax.ShapeDtypeStruct((B, H), F32),
        scratch_shapes=[pltpu.VMEM((G, H), F32)],
    )(pts, nidx, *args)
    return out


# speedup vs baseline: 1.1242x; 1.0109x over previous
import functools
import math

import jax
import jax.numpy as jnp
from jax import lax
from jax.experimental import pallas as pl
from jax.experimental.pallas import tpu as pltpu
from jax.experimental.pallas import tpu_sc as plsc

N = 512
H = 512
K = 16
ND = 3
G = 4
NEG = -1e30
BIG = 3.0e38

F32 = jnp.float32
I32 = jnp.int32

_NC = 2
_NS = 16
_NW = _NC * _NS
_RPW = (G * N) // _NW
_NCHUNK = N // 16


def _dot(a, b):
    return jax.lax.dot_general(a, b, (((1,), (0,)), ((), ())),
                               preferred_element_type=F32)


def _dot_t(a, b):
    return jax.lax.dot_general(a, b, (((1,), (1,)), ((), ())),
                               preferred_element_type=F32)


def _dot_bf(a, b):
    return jax.lax.dot_general(a.astype(jnp.bfloat16),
                               b.astype(jnp.bfloat16),
                               (((1,), (0,)), ((), ())),
                               preferred_element_type=F32)


def _layernorm(x, scale, bias, eps=1e-6):
    mu = jnp.mean(x, axis=-1, keepdims=True)
    var = jnp.mean(jnp.square(x - mu), axis=-1, keepdims=True)
    return (x - mu) * jax.lax.rsqrt(var + eps) * scale + bias


def _sc_knn_body(posT_hbm, out_hbm, pos_v, ibuf_v):
    wid = lax.axis_index("s") * _NC + lax.axis_index("c")
    g = wid // (N // _RPW)
    nbase = (wid % (N // _RPW)) * _RPW

    pltpu.sync_copy(posT_hbm.at[g], pos_v)

    iota16 = lax.iota(I32, 16)
    _gdn = lax.GatherDimensionNumbers(offset_dims=(), collapsed_slice_dims=(0,),
                                      start_index_map=(0,))

    def _bcast_lane(vec, lane):
        return lax.gather(vec, lane[:, None], _gdn, (1,),
                          mode=lax.GatherScatterMode.PROMISE_IN_BOUNDS)

    def one_row(n):
        cb = (n // 16) * 16
        lane = jnp.full((16,), n - cb, I32)
        pxn = _bcast_lane(pos_v[pl.ds(cb, 16)], lane)
        pyn = _bcast_lane(pos_v[pl.ds(N + cb, 16)], lane)
        pzn = _bcast_lane(pos_v[pl.ds(2 * N + cb, 16)], lane)

        def chunk_sorted(c):
            dx = pos_v[pl.ds(c * 16, 16)] - pxn
            dy = pos_v[pl.ds(N + c * 16, 16)] - pyn
            dz = pos_v[pl.ds(2 * N + c * 16, 16)] - pzn
            d2 = dx * dx + dy * dy + dz * dz
            return lax.sort((d2, iota16 + (c * 16)), num_keys=1)

        def merge(a, b):
            ak, ai = a
            bk, bi = b
            rbk = lax.rev(bk, (0,))
            rbi = lax.rev(bi, (0,))
            take = ak <= rbk
            mk = jnp.where(take, ak, rbk)
            mi = jnp.where(take, ai, rbi)
            return lax.sort((mk, mi), num_keys=1)

        def topk_range(c0, c1):
            if c1 - c0 == 1:
                return chunk_sorted(c0)
            mid = (c0 + c1) // 2
            return merge(topk_range(c0, mid), topk_range(mid, c1))

        _, bidx = topk_range(0, _NCHUNK)
        return bidx

    def row_body(r, _):
        ibuf_v[pl.ds(r * 128, K)] = one_row(nbase + r)
        return 0

    lax.fori_loop(0, _RPW, row_body, 0)
    pltpu.sync_copy(ibuf_v, out_hbm.at[pl.ds(wid * (_RPW * 128), _RPW * 128)])


def _sc_knn(posT):
    mesh = plsc.VectorSubcoreMesh(core_axis_name="c", subcore_axis_name="s",
                                  num_cores=_NC, num_subcores=_NS)
    fn = pl.kernel(
        _sc_knn_body,
        out_type=jax.ShapeDtypeStruct((G * N * 128,), I32),
        mesh=mesh,
        compiler_params=pltpu.CompilerParams(needs_layout_passes=False),
        scratch_types=[
            pltpu.VMEM((ND * N,), F32),
            pltpu.VMEM((_RPW * 128,), I32),
        ],
    )
    return fn(posT)


def _pt_layer(x, pos, M, Wq, bq, Wk, bk, Wv, bv, Wpe, bpe, Wpd, bpd, wa, ba,
              Wo, bo, lns, lnb):
    q = _dot(x, Wq) + bq
    k = _dot(x, Wk) + bk
    v = _dot(x, Wv) + bv
    pe = _dot(pos, Wpe) + bpe
    qq = q + pe
    qw = qq * wa
    kpe = k + pe
    u = _dot_t(qw, Wpd)
    c = (jnp.sum(qw * bpd, axis=1, keepdims=True) + ba
         - jnp.sum(u * pos, axis=1, keepdims=True))
    L = _dot_t(qw, kpe) + _dot_t(u, pos) + c
    Lm = jnp.where(M, L, NEG)
    rmax = jnp.max(Lm, axis=1, keepdims=True)
    e = jnp.exp(Lm - rmax)
    A = e / jnp.sum(e, axis=1, keepdims=True)
    out = _dot(A, v)
    y = jax.nn.relu(_dot(out, Wo) + bo)
    x = x + y
    return _layernorm(x, lns, lnb)


def _main_kernel(pts_ref, idx_ref, *rest):
    W0, b0 = rest[0], rest[1]
    l0 = rest[2:18]
    l1 = rest[18:34]
    (eWk, ebk, eWq, ebq, eWv, ebv, eWo1, ebo1, eWo2, ebo2, elns, elnb
     ) = rest[34:46]
    out_ref, xp_ref = rest[46], rest[47]
    g = pl.program_id(0)

    @pl.when(g < G)
    def _cloud():
        pts = pts_ref[0]
        pos = pts[:, :ND]
        idx = idx_ref[0][:, :K].astype(jnp.int16)
        iota = jax.lax.broadcasted_iota(jnp.int16, (N, N), 1)
        M = jnp.zeros((N, N), jnp.bool_)
        for j in range(K):
            M = jnp.logical_or(M, iota == idx[:, j:j + 1])

        x = _dot(pts, W0[...]) + b0[...]
        x = _pt_layer(x, pos, M, *(w[...] for w in l0))
        x = _pt_layer(x, pos, M, *(w[...] for w in l1))
        xp_ref[pl.ds(g, 1), :] = jnp.max(x, axis=0, keepdims=True)

    @pl.when(g == G)
    def _enc():
        scale = 1.0 / math.sqrt(float(H))
        for b in range(G // 2):
            xb = xp_ref[2 * b:2 * b + 2, :]
            k = _dot(xb, eWk[...]) + ebk[...]
            q = _dot(xb, eWq[...]) + ebq[...]
            v = _dot(xb, eWv[...]) + ebv[...]
            attn = _dot_t(q, k) * scale
            attn = attn - jnp.max(attn, axis=1, keepdims=True)
            e = jnp.exp(attn)
            attn = e / jnp.sum(e, axis=1, keepdims=True)
            out = _dot(attn, v)
            out = jax.nn.relu(_dot(out, eWo1[...]) + ebo1[...])
            out = _dot(out, eWo2[...]) + ebo2[...]
            xo = _layernorm(xb + out, elns[...], elnb[...])
            out_ref[b] = jnp.max(xo, axis=0)


def _row(a):
    return a.reshape(1, -1)


@jax.jit
def kernel(points, params):
    B, S, Np, C = points.shape
    pts = points.reshape(G, Np, C)
    posT = jnp.swapaxes(pts, 1, 2)[:, :ND, :].reshape(G, ND * Np)

    nidx = _sc_knn(posT).reshape(G, Np, 128)

    p = params
    args = [p['W0'], _row(p['b0'])]
    for i in range(2):
        lp = p['layer%d' % i]
        args += [
            lp['Wq'], _row(lp['bq']),
            lp['Wk'], _row(lp['bk']),
            lp['Wv'], _row(lp['bv']),
            lp['Wpe'], _row(lp['bpe']),
            lp['Wpd'], _row(lp['bpd']),
            lp['Wa'].reshape(1, H), lp['ba'].reshape(1, 1),
            lp['Wo'], _row(lp['bo']),
            _row(lp['ln_scale']), _row(lp['ln_bias']),
        ]

    ep = p['enc']
    args += [ep['Wk'], _row(ep['bk']), ep['Wq'], _row(ep['bq']),
             ep['Wv'], _row(ep['bv']), ep['Wo1'], _row(ep['bo1']),
             ep['Wo2'], _row(ep['bo2']),
             _row(ep['ln_scale']), _row(ep['ln_bias'])]

    rep = [pl.BlockSpec(a.shape, lambda g, nd=a.ndim: (0,) * nd) for a in args]
    clip = lambda g: (jnp.minimum(g, G - 1), 0, 0)
    out = pl.pallas_call(
        _main_kernel,
        grid=(G + 1,),
        in_specs=[
            pl.BlockSpec((1, Np, C), clip),
            pl.BlockSpec((1, Np, 128), clip),
        ] + rep,
        out_specs=pl.BlockSpec((B, H), lambda g: (0, 0)),
        out_shape=jax.ShapeDtypeStruct((B, H), F32),
        scratch_shapes=[pltpu.VMEM((G, H), F32)],
    )(pts, nidx, *args)
    return out
